# baseline (speedup 1.0000x reference)
import functools
import jax
import jax.numpy as jnp
from jax import lax
from jax.experimental import pallas as pl
from jax.experimental.pallas import tpu as pltpu
from jax.experimental.pallas import tpu_sc as plsc

N_VARS = 64
EMBED_DIM = 64
_FLAT = N_VARS * EMBED_DIM
_B = 16
_NP = 256
_N_WORKERS = 32
_F_PER_W = _FLAT // _N_WORKERS


def _sc_body(emb_hbm, out_hbm, vals, buf, sem):
    wid = lax.axis_index("s") * 2 + lax.axis_index("c")
    fbase = wid * _F_PER_W
    pltpu.sync_copy(emb_hbm.at[pl.ds(fbase, _F_PER_W)], vals)

    for g in range(_F_PER_W // 16):
        v16 = vals[pl.ds(g * 16, 16)]
        for k in range(16):
            splat = lax.broadcast(v16[k], (16,))
            for t in range(_NP // 16):
                buf[g * 16 + k, pl.ds(t * 16, 16)] = splat

    copies = []
    for b in range(_B):
        copies.append(
            pltpu.async_copy(
                buf, out_hbm.at[pl.ds(b * _FLAT + fbase, _F_PER_W), :], sem
            )
        )
    for c in copies:
        c.wait()


def kernel(x, channel_emb):
    B, n_patches, _ = x.shape
    emb_flat = channel_emb.reshape(_FLAT)
    sc_kernel = functools.partial(
        pl.kernel,
        mesh=plsc.VectorSubcoreMesh(core_axis_name="c", subcore_axis_name="s"),
        out_type=jax.ShapeDtypeStruct((_B * _FLAT, _NP), channel_emb.dtype),
        scratch_types=[
            pltpu.VMEM((_F_PER_W,), channel_emb.dtype),
            pltpu.VMEM((_F_PER_W, _NP), channel_emb.dtype),
            pltpu.SemaphoreType.DMA,
        ],
    )(_sc_body)
    out2d = sc_kernel(emb_flat)
    out_t = out2d.reshape(B, N_VARS, EMBED_DIM, n_patches)
    return out_t.transpose(0, 3, 1, 2)


# speedup vs baseline: 1.9798x; 1.9798x over previous
import jax
import jax.numpy as jnp
from jax.experimental import pallas as pl
from jax.experimental.pallas import tpu as pltpu

N_VARS = 64
EMBED_DIM = 64
_NSEM = 8
_NQ = 4


def _bcast_kernel(emb_ref, out_ref, scratch_ref, sems):
    emb_t = jnp.transpose(emb_ref[...], (1, 0))
    n_patches = out_ref.shape[1]
    flat = scratch_ref.shape[0]
    B = out_ref.shape[0] // flat
    v_per_q = N_VARS // _NQ
    rows_per_q = flat // _NQ
    copies = []
    for q in range(_NQ):
        for v in range(q * v_per_q, (q + 1) * v_per_q):
            scratch_ref[pl.ds(v * EMBED_DIM, EMBED_DIM), :] = jnp.broadcast_to(
                emb_t[:, v : v + 1], (EMBED_DIM, n_patches)
            )
        qbase = q * rows_per_q
        for b in range(B):
            c = pltpu.make_async_copy(
                scratch_ref.at[pl.ds(qbase, rows_per_q), :],
                out_ref.at[pl.ds(b * flat + qbase, rows_per_q), :],
                sems.at[(q * B + b) % _NSEM],
            )
            c.start()
            copies.append(c)
    for c in copies:
        c.wait()


def kernel(x, channel_emb):
    B, n_patches, _ = x.shape
    flat = N_VARS * EMBED_DIM
    out2d = pl.pallas_call(
        _bcast_kernel,
        in_specs=[pl.BlockSpec(memory_space=pltpu.VMEM)],
        out_specs=pl.BlockSpec(memory_space=pl.ANY),
        out_shape=jax.ShapeDtypeStruct((B * flat, n_patches), channel_emb.dtype),
        scratch_shapes=[
            pltpu.VMEM((flat, n_patches), channel_emb.dtype),
            pltpu.SemaphoreType.DMA((_NSEM,)),
        ],
    )(channel_emb)
    out_t = out2d.reshape(B, N_VARS, EMBED_DIM, n_patches)
    return out_t.transpose(0, 3, 1, 2)
